# baseline (speedup 1.0000x reference)
import functools

import jax
import jax.numpy as jnp
from jax import lax
from jax.experimental import pallas as pl
from jax.experimental.pallas import tpu as pltpu
from jax.experimental.pallas import tpu_sc as plsc

N = 10000
E = 320000
D = 128

NC = 2
NS = 16
NW = NC * NS

CHUNK = 128
BLK = 32
NB = 5
HB = BLK // 2
CH = NB * BLK
EPT = CH * CHUNK
EP = NS * EPT
RPT = 632
NP = NS * RPT
DUMMY = N

BN = 2000
GRID = N // BN

f32 = jnp.float32
_PREC = lax.Precision.HIGHEST

_mesh = plsc.VectorSubcoreMesh(
    core_axis_name="c", subcore_axis_name="s", num_cores=NC, num_subcores=NS)


@functools.partial(
    pl.kernel,
    out_type=jax.ShapeDtypeStruct((NC * NP, D), f32),
    mesh=_mesh,
    scratch_types=[
        pltpu.VMEM((BLK + 8, CHUNK), jnp.int32),
        pltpu.VMEM((BLK, CHUNK), jnp.int32),
        pltpu.VMEM((CHUNK, D), f32),
        pltpu.VMEM((CHUNK, D), f32),
        pltpu.VMEM_SHARED((NP, D), f32),
        pltpu.SemaphoreType.DMA,
        pltpu.SemaphoreType.DMA,
        pltpu.SemaphoreType.DMA,
        pltpu.SemaphoreType.DMA,
    ],
)
def _scatter2(table, gidx, sidx, out, gv, sv, rb0, rb1, acc,
              sg0, sg1, ss0, ss1):
    c = lax.axis_index("c")
    s = lax.axis_index("s")
    w = c * NS + s

    def _zrow(i, carry):
        for k in range(D // 16):
            rb0[i, pl.ds(k * 16, 16)] = jnp.zeros((16,), f32)
        return carry
    lax.fori_loop(0, CHUNK, _zrow, 0)

    base = s * RPT
    for k in range(RPT // CHUNK):
        pltpu.sync_copy(rb0, acc.at[pl.ds(base + k * CHUNK, CHUNK)])
    rem = RPT % CHUNK
    pltpu.sync_copy(rb0.at[pl.ds(0, rem)],
                    acc.at[pl.ds(base + (RPT // CHUNK) * CHUNK, rem)])

    plsc.subcore_barrier()

    def _blk(b, carry):
        st = (w * NB + b) * BLK
        pltpu.sync_copy(gidx.at[pl.ds(st, BLK + 8)], gv)
        pltpu.sync_copy(sidx.at[pl.ds(st, BLK)], sv)

        pltpu.async_copy(table.at[gv.at[0]], rb0, sg0).wait()
        pltpu.async_copy(table.at[gv.at[1]], rb1, sg1)
        pltpu.async_copy(rb0, acc.at[sv.at[0]], ss0, add=True)
        pltpu.make_async_copy(table.at[gv.at[1]], rb1, sg1).wait()
        pltpu.make_async_copy(rb0, acc.at[sv.at[0]], ss0).wait()
        pltpu.async_copy(table.at[gv.at[2]], rb0, sg0)
        pltpu.async_copy(rb1, acc.at[sv.at[1]], ss1, add=True)

        def _pair(i, carry2):
            j0 = 2 * i
            j1 = j0 + 1
            pltpu.make_async_copy(table.at[gv.at[j0]], rb0, sg0).wait()
            pltpu.make_async_copy(rb1, acc.at[sv.at[j0]], ss1).wait()
            pltpu.async_copy(table.at[gv.at[j1]], rb1, sg1)
            pltpu.async_copy(rb0, acc.at[sv.at[j0]], ss0, add=True)
            pltpu.make_async_copy(table.at[gv.at[j1]], rb1, sg1).wait()
            pltpu.make_async_copy(rb0, acc.at[sv.at[j0]], ss0).wait()
            pltpu.async_copy(table.at[gv.at[j0 + 2]], rb0, sg0)
            pltpu.async_copy(rb1, acc.at[sv.at[j1]], ss1, add=True)
            return carry2
        lax.fori_loop(1, HB, _pair, 0)

        pltpu.make_async_copy(table.at[gv.at[BLK]], rb0, sg0).wait()
        pltpu.make_async_copy(rb1, acc.at[sv.at[BLK - 1]], ss1).wait()
        return carry
    lax.fori_loop(0, NB, _blk, 0)

    plsc.subcore_barrier()
    pltpu.sync_copy(acc.at[pl.ds(base, RPT)],
                    out.at[pl.ds(c * NP + base, RPT)])


@functools.partial(
    pl.kernel,
    out_type=jax.ShapeDtypeStruct((NC * NP, D), f32),
    mesh=_mesh,
    scratch_types=[
        pltpu.VMEM((BLK, CHUNK), jnp.int32),
        pltpu.VMEM((CHUNK, D), f32),
        pltpu.VMEM_SHARED((NP, D), f32),
    ],
)
def _counts(sidx, out, sv, obuf, acc):
    c = lax.axis_index("c")
    s = lax.axis_index("s")
    w = c * NS + s

    def _fill(val):
        def _row(i, carry):
            for k in range(D // 16):
                obuf[i, pl.ds(k * 16, 16)] = jnp.full((16,), val, f32)
            return carry
        lax.fori_loop(0, CHUNK, _row, 0)

    _fill(0.0)
    base = s * RPT
    for k in range(RPT // CHUNK):
        pltpu.sync_copy(obuf, acc.at[pl.ds(base + k * CHUNK, CHUNK)])
    rem = RPT % CHUNK
    pltpu.sync_copy(obuf.at[pl.ds(0, rem)],
                    acc.at[pl.ds(base + (RPT // CHUNK) * CHUNK, rem)])
    _fill(1.0)

    plsc.subcore_barrier()

    def _blk(b, carry):
        pltpu.sync_copy(sidx.at[pl.ds((w * NB + b) * BLK, BLK)], sv)

        def _body(j, carry2):
            pltpu.sync_copy(obuf, acc.at[sv.at[j]], add=True)
            return carry2
        return lax.fori_loop(0, BLK, _body, carry)
    lax.fori_loop(0, NB, _blk, 0)

    plsc.subcore_barrier()
    pltpu.sync_copy(acc.at[pl.ds(base, RPT)],
                    out.at[pl.ds(c * NP + base, RPT)])


def _pre_body(h_ref, w12_ref, wr_ref, br_ref, a12_ref, hr_ref):
    h = h_ref[...]
    a12_ref[0] = jnp.dot(h, w12_ref[0], preferred_element_type=f32,
                         precision=_PREC)
    a12_ref[1] = jnp.dot(h, w12_ref[1], preferred_element_type=f32,
                         precision=_PREC)
    hr_ref[...] = jnp.dot(h, wr_ref[...], preferred_element_type=f32,
                          precision=_PREC) + br_ref[...]


_pre = pl.pallas_call(
    _pre_body,
    grid=(GRID,),
    in_specs=[
        pl.BlockSpec((BN, D), lambda i: (i, 0)),
        pl.BlockSpec((2, D, D), lambda i: (0, 0, 0)),
        pl.BlockSpec((D, D), lambda i: (0, 0)),
        pl.BlockSpec((1, D), lambda i: (0, 0)),
    ],
    out_specs=[
        pl.BlockSpec((2, BN, D), lambda i: (0, i, 0)),
        pl.BlockSpec((BN, D), lambda i: (i, 0)),
    ],
    out_shape=[
        jax.ShapeDtypeStruct((2, N, D), f32),
        jax.ShapeDtypeStruct((N, D), f32),
    ],
)


def _comb_body(hr_ref, accs_ref, cd_ref, cs_ref, o_ref):
    rd = 1.0 / jnp.maximum(cd_ref[...], 1.0)
    rs = 1.0 / jnp.maximum(cs_ref[...], 1.0)
    v = hr_ref[...] + accs_ref[0] * rd + accs_ref[1] * rs
    o_ref[...] = jnp.maximum(v, 0.0)


_comb = pl.pallas_call(
    _comb_body,
    grid=(GRID,),
    in_specs=[
        pl.BlockSpec((BN, D), lambda i: (i, 0)),
        pl.BlockSpec((2, BN, D), lambda i: (0, i, 0)),
        pl.BlockSpec((BN, 1), lambda i: (i, 0)),
        pl.BlockSpec((BN, 1), lambda i: (i, 0)),
    ],
    out_specs=pl.BlockSpec((BN, D), lambda i: (i, 0)),
    out_shape=jax.ShapeDtypeStruct((N, D), f32),
)


def _fin_body(x_ref, h1_ref, h2_ref, h3_ref, wf_ref, bf_ref, o_ref):
    acc = jnp.dot(x_ref[...], wf_ref[0:D], preferred_element_type=f32,
                  precision=_PREC)
    acc += jnp.dot(h1_ref[...], wf_ref[D:2 * D], preferred_element_type=f32,
                   precision=_PREC)
    acc += jnp.dot(h2_ref[...], wf_ref[2 * D:3 * D], preferred_element_type=f32,
                   precision=_PREC)
    acc += jnp.dot(h3_ref[...], wf_ref[3 * D:4 * D], preferred_element_type=f32,
                   precision=_PREC)
    o_ref[...] = acc + bf_ref[...]


_fin = pl.pallas_call(
    _fin_body,
    grid=(GRID,),
    in_specs=[
        pl.BlockSpec((BN, D), lambda i: (i, 0)),
        pl.BlockSpec((BN, D), lambda i: (i, 0)),
        pl.BlockSpec((BN, D), lambda i: (i, 0)),
        pl.BlockSpec((BN, D), lambda i: (i, 0)),
        pl.BlockSpec((4 * D, D), lambda i: (0, 0)),
        pl.BlockSpec((1, D), lambda i: (0, 0)),
    ],
    out_specs=pl.BlockSpec((BN, D), lambda i: (i, 0)),
    out_shape=jax.ShapeDtypeStruct((N, D), f32),
)


def kernel(x, edge_index, W1_0, W2_0, Wr_0, br_0, W1_1, W2_1, Wr_1, br_1,
           W1_2, W2_2, Wr_2, br_2, Wf, bf):
    src = edge_index[0]
    dst = edge_index[1]
    pad = EP - E
    padz = jnp.zeros((pad,), jnp.int32)
    padd = jnp.full((pad,), DUMMY, jnp.int32)

    gidx = jnp.stack([
        jnp.concatenate([src, padz]),
        jnp.concatenate([dst + N, padz]),
    ]).reshape(NW * CH, CHUNK)
    gidx = jnp.concatenate([gidx, jnp.zeros((8, CHUNK), jnp.int32)])
    sidx = jnp.stack([
        jnp.concatenate([dst, padd]),
        jnp.concatenate([src, padd]),
    ]).reshape(NW * CH, CHUNK)

    cnt = _counts(sidx)
    cd = cnt[0:N, 0:1]
    cs = cnt[NP:NP + N, 0:1]

    params = [(W1_0, W2_0, Wr_0, br_0), (W1_1, W2_1, Wr_1, br_1),
              (W1_2, W2_2, Wr_2, br_2)]
    hs = [x]
    h = x
    for (W1, W2, Wr, br) in params:
        a12, hr = _pre(h, jnp.stack([W1, W2]), Wr, br.reshape(1, D))
        accs = _scatter2(a12.reshape(2 * N, D), gidx, sidx)
        h = _comb(hr, accs.reshape(NC, NP, D), cd, cs)
        hs.append(h)

    return _fin(hs[0], hs[1], hs[2], hs[3], Wf, bf.reshape(1, D))


# speedup vs baseline: 1.0273x; 1.0273x over previous
import functools

import jax
import jax.numpy as jnp
from jax import lax
from jax.experimental import pallas as pl
from jax.experimental.pallas import tpu as pltpu
from jax.experimental.pallas import tpu_sc as plsc

N = 10000
E = 320000
D = 128

NC = 2
NS = 16
NW = NC * NS

CHUNK = 128
BLK = 8
NB = 20
CH = NB * BLK
EPT = CH * CHUNK
EP = NS * EPT
RPT = 632
NP = NS * RPT
DUMMY = N

BN = 2000
GRID = N // BN

f32 = jnp.float32
_PREC = lax.Precision.HIGHEST

_mesh = plsc.VectorSubcoreMesh(
    core_axis_name="c", subcore_axis_name="s", num_cores=NC, num_subcores=NS)


@functools.partial(
    pl.kernel,
    out_type=jax.ShapeDtypeStruct((NC * NP, D), f32),
    mesh=_mesh,
    scratch_types=[
        pltpu.VMEM((BLK, CHUNK), jnp.int32),
        pltpu.VMEM((BLK, CHUNK), jnp.int32),
        pltpu.VMEM((CHUNK, D), f32),
        pltpu.VMEM((CHUNK, D), f32),
        pltpu.VMEM_SHARED((NP, D), f32),
        pltpu.SemaphoreType.DMA,
        pltpu.SemaphoreType.DMA,
        pltpu.SemaphoreType.DMA,
        pltpu.SemaphoreType.DMA,
    ],
)
def _scatter2(table, gidx, sidx, out, gv, sv, rb0, rb1, acc,
              sg0, sg1, ss0, ss1):
    c = lax.axis_index("c")
    s = lax.axis_index("s")
    w = c * NS + s

    def _zrow(i, carry):
        for k in range(D // 16):
            rb0[i, pl.ds(k * 16, 16)] = jnp.zeros((16,), f32)
        return carry
    lax.fori_loop(0, CHUNK, _zrow, 0)

    base = s * RPT
    for k in range(RPT // CHUNK):
        pltpu.sync_copy(rb0, acc.at[pl.ds(base + k * CHUNK, CHUNK)])
    rem = RPT % CHUNK
    pltpu.sync_copy(rb0.at[pl.ds(0, rem)],
                    acc.at[pl.ds(base + (RPT // CHUNK) * CHUNK, rem)])

    plsc.subcore_barrier()

    rbs = (rb0, rb1)
    sgs = (sg0, sg1)
    sss = (ss0, ss1)

    def _blk(b, carry):
        st = (w * NB + b) * BLK
        pltpu.sync_copy(gidx.at[pl.ds(st, BLK)], gv)
        pltpu.sync_copy(sidx.at[pl.ds(st, BLK)], sv)
        dg = [None, None]
        dsc = [None, None]
        for k in range(BLK + 1):
            p = k & 1
            if k < BLK:
                if dsc[p] is not None:
                    dsc[p].wait()
                dg[p] = pltpu.async_copy(table.at[gv.at[k]], rbs[p], sgs[p])
            if k >= 1:
                q = (k - 1) & 1
                dg[q].wait()
                dsc[q] = pltpu.async_copy(
                    rbs[q], acc.at[sv.at[k - 1]], sss[q], add=True)
        for p in range(2):
            if dsc[p] is not None:
                dsc[p].wait()
        return carry
    lax.fori_loop(0, NB, _blk, 0)

    plsc.subcore_barrier()
    pltpu.sync_copy(acc.at[pl.ds(base, RPT)],
                    out.at[pl.ds(c * NP + base, RPT)])


@functools.partial(
    pl.kernel,
    out_type=jax.ShapeDtypeStruct((NC * NP, D), f32),
    mesh=_mesh,
    scratch_types=[
        pltpu.VMEM((BLK, CHUNK), jnp.int32),
        pltpu.VMEM((CHUNK, D), f32),
        pltpu.VMEM_SHARED((NP, D), f32),
    ],
)
def _counts(sidx, out, sv, obuf, acc):
    c = lax.axis_index("c")
    s = lax.axis_index("s")
    w = c * NS + s

    def _fill(val):
        def _row(i, carry):
            for k in range(D // 16):
                obuf[i, pl.ds(k * 16, 16)] = jnp.full((16,), val, f32)
            return carry
        lax.fori_loop(0, CHUNK, _row, 0)

    _fill(0.0)
    base = s * RPT
    for k in range(RPT // CHUNK):
        pltpu.sync_copy(obuf, acc.at[pl.ds(base + k * CHUNK, CHUNK)])
    rem = RPT % CHUNK
    pltpu.sync_copy(obuf.at[pl.ds(0, rem)],
                    acc.at[pl.ds(base + (RPT // CHUNK) * CHUNK, rem)])
    _fill(1.0)

    plsc.subcore_barrier()

    def _blk(b, carry):
        pltpu.sync_copy(sidx.at[pl.ds((w * NB + b) * BLK, BLK)], sv)

        def _body(j, carry2):
            pltpu.sync_copy(obuf, acc.at[sv.at[j]], add=True)
            return carry2
        return lax.fori_loop(0, BLK, _body, carry)
    lax.fori_loop(0, NB, _blk, 0)

    plsc.subcore_barrier()
    pltpu.sync_copy(acc.at[pl.ds(base, RPT)],
                    out.at[pl.ds(c * NP + base, RPT)])


def _pre_body(h_ref, w12_ref, wr_ref, br_ref, a12_ref, hr_ref):
    h = h_ref[...]
    a12_ref[0] = jnp.dot(h, w12_ref[0], preferred_element_type=f32,
                         precision=_PREC)
    a12_ref[1] = jnp.dot(h, w12_ref[1], preferred_element_type=f32,
                         precision=_PREC)
    hr_ref[...] = jnp.dot(h, wr_ref[...], preferred_element_type=f32,
                          precision=_PREC) + br_ref[...]


_pre = pl.pallas_call(
    _pre_body,
    grid=(GRID,),
    in_specs=[
        pl.BlockSpec((BN, D), lambda i: (i, 0)),
        pl.BlockSpec((2, D, D), lambda i: (0, 0, 0)),
        pl.BlockSpec((D, D), lambda i: (0, 0)),
        pl.BlockSpec((1, D), lambda i: (0, 0)),
    ],
    out_specs=[
        pl.BlockSpec((2, BN, D), lambda i: (0, i, 0)),
        pl.BlockSpec((BN, D), lambda i: (i, 0)),
    ],
    out_shape=[
        jax.ShapeDtypeStruct((2, N, D), f32),
        jax.ShapeDtypeStruct((N, D), f32),
    ],
)


def _comb_body(hr_ref, accs_ref, cd_ref, cs_ref, o_ref):
    rd = 1.0 / jnp.maximum(cd_ref[...], 1.0)
    rs = 1.0 / jnp.maximum(cs_ref[...], 1.0)
    v = hr_ref[...] + accs_ref[0] * rd + accs_ref[1] * rs
    o_ref[...] = jnp.maximum(v, 0.0)


_comb = pl.pallas_call(
    _comb_body,
    grid=(GRID,),
    in_specs=[
        pl.BlockSpec((BN, D), lambda i: (i, 0)),
        pl.BlockSpec((2, BN, D), lambda i: (0, i, 0)),
        pl.BlockSpec((BN, 1), lambda i: (i, 0)),
        pl.BlockSpec((BN, 1), lambda i: (i, 0)),
    ],
    out_specs=pl.BlockSpec((BN, D), lambda i: (i, 0)),
    out_shape=jax.ShapeDtypeStruct((N, D), f32),
)


def _fin_body(x_ref, h1_ref, h2_ref, h3_ref, wf_ref, bf_ref, o_ref):
    acc = jnp.dot(x_ref[...], wf_ref[0:D], preferred_element_type=f32,
                  precision=_PREC)
    acc += jnp.dot(h1_ref[...], wf_ref[D:2 * D], preferred_element_type=f32,
                   precision=_PREC)
    acc += jnp.dot(h2_ref[...], wf_ref[2 * D:3 * D], preferred_element_type=f32,
                   precision=_PREC)
    acc += jnp.dot(h3_ref[...], wf_ref[3 * D:4 * D], preferred_element_type=f32,
                   precision=_PREC)
    o_ref[...] = acc + bf_ref[...]


_fin = pl.pallas_call(
    _fin_body,
    grid=(GRID,),
    in_specs=[
        pl.BlockSpec((BN, D), lambda i: (i, 0)),
        pl.BlockSpec((BN, D), lambda i: (i, 0)),
        pl.BlockSpec((BN, D), lambda i: (i, 0)),
        pl.BlockSpec((BN, D), lambda i: (i, 0)),
        pl.BlockSpec((4 * D, D), lambda i: (0, 0)),
        pl.BlockSpec((1, D), lambda i: (0, 0)),
    ],
    out_specs=pl.BlockSpec((BN, D), lambda i: (i, 0)),
    out_shape=jax.ShapeDtypeStruct((N, D), f32),
)


def kernel(x, edge_index, W1_0, W2_0, Wr_0, br_0, W1_1, W2_1, Wr_1, br_1,
           W1_2, W2_2, Wr_2, br_2, Wf, bf):
    src = edge_index[0]
    dst = edge_index[1]
    pad = EP - E
    padz = jnp.zeros((pad,), jnp.int32)
    padd = jnp.full((pad,), DUMMY, jnp.int32)

    gidx = jnp.stack([
        jnp.concatenate([src, padz]),
        jnp.concatenate([dst + N, padz]),
    ]).reshape(NW * CH, CHUNK)
    sidx = jnp.stack([
        jnp.concatenate([dst, padd]),
        jnp.concatenate([src, padd]),
    ]).reshape(NW * CH, CHUNK)

    cnt = _counts(sidx)
    cd = cnt[0:N, 0:1]
    cs = cnt[NP:NP + N, 0:1]

    params = [(W1_0, W2_0, Wr_0, br_0), (W1_1, W2_1, Wr_1, br_1),
              (W1_2, W2_2, Wr_2, br_2)]
    hs = [x]
    h = x
    for (W1, W2, Wr, br) in params:
        a12, hr = _pre(h, jnp.stack([W1, W2]), Wr, br.reshape(1, D))
        accs = _scatter2(a12.reshape(2 * N, D), gidx, sidx)
        h = _comb(hr, accs.reshape(NC, NP, D), cd, cs)
        hs.append(h)

    return _fin(hs[0], hs[1], hs[2], hs[3], Wf, bf.reshape(1, D))


# speedup vs baseline: 2.7114x; 2.6393x over previous
import functools

import jax
import jax.numpy as jnp
from jax import lax
from jax.experimental import pallas as pl
from jax.experimental.pallas import tpu as pltpu
from jax.experimental.pallas import tpu_sc as plsc

N = 10000
E = 320000
D = 128

NC = 2
NS = 16
NW = NC * NS

CHUNK = 128
BLK = 8
NB = 20
CH = NB * BLK
EPT = CH * CHUNK
EP = NS * EPT
RPT = 632
NP = NS * RPT
DUMMY = N

BN = 2000
GRID = N // BN

f32 = jnp.float32
_PREC = lax.Precision.HIGHEST

_mesh = plsc.VectorSubcoreMesh(
    core_axis_name="c", subcore_axis_name="s", num_cores=NC, num_subcores=NS)


@functools.partial(
    pl.kernel,
    out_type=jax.ShapeDtypeStruct((NC * NP, D), f32),
    mesh=_mesh,
    scratch_types=[
        pltpu.VMEM((BLK, CHUNK), jnp.int32),
        pltpu.VMEM((BLK, CHUNK), jnp.int32),
        pltpu.VMEM((CHUNK, D), f32),
        pltpu.VMEM((CHUNK, D), f32),
        pltpu.VMEM_SHARED((NP, D), f32),
        pltpu.SemaphoreType.DMA,
        pltpu.SemaphoreType.DMA,
        pltpu.SemaphoreType.DMA,
        pltpu.SemaphoreType.DMA,
    ],
)
def _scatter2(table, gidx, sidx, out, gv, sv, rb0, rb1, acc,
              sg0, sg1, ss0, ss1):
    c = lax.axis_index("c")
    s = lax.axis_index("s")
    w = c * NS + s

    def _zrow(i, carry):
        for k in range(D // 16):
            rb0[i, pl.ds(k * 16, 16)] = jnp.zeros((16,), f32)
        return carry
    lax.fori_loop(0, CHUNK, _zrow, 0)

    base = s * RPT
    for k in range(RPT // CHUNK):
        pltpu.sync_copy(rb0, acc.at[pl.ds(base + k * CHUNK, CHUNK)])
    rem = RPT % CHUNK
    pltpu.sync_copy(rb0.at[pl.ds(0, rem)],
                    acc.at[pl.ds(base + (RPT // CHUNK) * CHUNK, rem)])

    plsc.subcore_barrier()

    rbs = (rb0, rb1)
    sgs = (sg0, sg1)
    sss = (ss0, ss1)

    def _blk(b, carry):
        st = (w * NB + b) * BLK
        pltpu.sync_copy(gidx.at[pl.ds(st, BLK)], gv)
        pltpu.sync_copy(sidx.at[pl.ds(st, BLK)], sv)
        dg = [None, None]
        dsc = [None, None]
        for k in range(BLK + 1):
            p = k & 1
            if k < BLK:
                if dsc[p] is not None:
                    dsc[p].wait()
                dg[p] = pltpu.async_copy(table.at[gv.at[k]], rbs[p], sgs[p])
            if k >= 1:
                q = (k - 1) & 1
                dg[q].wait()
                dsc[q] = pltpu.async_copy(
                    rbs[q], acc.at[sv.at[k - 1]], sss[q], add=True)
        for p in range(2):
            if dsc[p] is not None:
                dsc[p].wait()
        return carry
    lax.fori_loop(0, NB, _blk, 0)

    plsc.subcore_barrier()
    pltpu.sync_copy(acc.at[pl.ds(base, RPT)],
                    out.at[pl.ds(c * NP + base, RPT)])


@functools.partial(
    pl.kernel,
    out_type=jax.ShapeDtypeStruct((NC * NP, D), f32),
    mesh=_mesh,
    scratch_types=[
        pltpu.VMEM((BLK, CHUNK), jnp.int32),
        pltpu.VMEM((CHUNK, D), f32),
        pltpu.VMEM_SHARED((NP, D), f32),
    ],
)
def _counts(sidx, out, sv, obuf, acc):
    c = lax.axis_index("c")
    s = lax.axis_index("s")
    w = c * NS + s

    def _fill(val):
        def _row(i, carry):
            for k in range(D // 16):
                obuf[i, pl.ds(k * 16, 16)] = jnp.full((16,), val, f32)
            return carry
        lax.fori_loop(0, CHUNK, _row, 0)

    _fill(0.0)
    base = s * RPT
    for k in range(RPT // CHUNK):
        pltpu.sync_copy(obuf, acc.at[pl.ds(base + k * CHUNK, CHUNK)])
    rem = RPT % CHUNK
    pltpu.sync_copy(obuf.at[pl.ds(0, rem)],
                    acc.at[pl.ds(base + (RPT // CHUNK) * CHUNK, rem)])
    _fill(1.0)

    plsc.subcore_barrier()

    def _blk(b, carry):
        pltpu.sync_copy(sidx.at[pl.ds((w * NB + b) * BLK, BLK)], sv)

        def _body(j, carry2):
            pltpu.sync_copy(obuf, acc.at[sv.at[j]], add=True)
            return carry2
        return lax.fori_loop(0, BLK, _body, carry)
    lax.fori_loop(0, NB, _blk, 0)

    plsc.subcore_barrier()
    pltpu.sync_copy(acc.at[pl.ds(base, RPT)],
                    out.at[pl.ds(c * NP + base, RPT)])


def _pre_body(h_ref, w12_ref, wr_ref, br_ref, a12_ref, hr_ref):
    h = h_ref[...]
    a12_ref[0] = jnp.dot(h, w12_ref[0], preferred_element_type=f32,
                         precision=_PREC)
    a12_ref[1] = jnp.dot(h, w12_ref[1], preferred_element_type=f32,
                         precision=_PREC)
    hr_ref[...] = jnp.dot(h, wr_ref[...], preferred_element_type=f32,
                          precision=_PREC) + br_ref[...]


_pre = pl.pallas_call(
    _pre_body,
    grid=(GRID,),
    in_specs=[
        pl.BlockSpec((BN, D), lambda i: (i, 0)),
        pl.BlockSpec((2, D, D), lambda i: (0, 0, 0)),
        pl.BlockSpec((D, D), lambda i: (0, 0)),
        pl.BlockSpec((1, D), lambda i: (0, 0)),
    ],
    out_specs=[
        pl.BlockSpec((2, BN, D), lambda i: (0, i, 0)),
        pl.BlockSpec((BN, D), lambda i: (i, 0)),
    ],
    out_shape=[
        jax.ShapeDtypeStruct((2, N, D), f32),
        jax.ShapeDtypeStruct((N, D), f32),
    ],
)


def _comb_body(hr_ref, accs_ref, cd_ref, cs_ref, o_ref):
    rd = 1.0 / jnp.maximum(cd_ref[...], 1.0)
    rs = 1.0 / jnp.maximum(cs_ref[...], 1.0)
    v = hr_ref[...] + accs_ref[0] * rd + accs_ref[1] * rs
    o_ref[...] = jnp.maximum(v, 0.0)


_comb = pl.pallas_call(
    _comb_body,
    grid=(GRID,),
    in_specs=[
        pl.BlockSpec((BN, D), lambda i: (i, 0)),
        pl.BlockSpec((2, BN, D), lambda i: (0, i, 0)),
        pl.BlockSpec((BN, 1), lambda i: (i, 0)),
        pl.BlockSpec((BN, 1), lambda i: (i, 0)),
    ],
    out_specs=pl.BlockSpec((BN, D), lambda i: (i, 0)),
    out_shape=jax.ShapeDtypeStruct((N, D), f32),
)


def _fin_body(x_ref, h1_ref, h2_ref, h3_ref, wf_ref, bf_ref, o_ref):
    acc = jnp.dot(x_ref[...], wf_ref[0:D], preferred_element_type=f32,
                  precision=_PREC)
    acc += jnp.dot(h1_ref[...], wf_ref[D:2 * D], preferred_element_type=f32,
                   precision=_PREC)
    acc += jnp.dot(h2_ref[...], wf_ref[2 * D:3 * D], preferred_element_type=f32,
                   precision=_PREC)
    acc += jnp.dot(h3_ref[...], wf_ref[3 * D:4 * D], preferred_element_type=f32,
                   precision=_PREC)
    o_ref[...] = acc + bf_ref[...]


_fin = pl.pallas_call(
    _fin_body,
    grid=(GRID,),
    in_specs=[
        pl.BlockSpec((BN, D), lambda i: (i, 0)),
        pl.BlockSpec((BN, D), lambda i: (i, 0)),
        pl.BlockSpec((BN, D), lambda i: (i, 0)),
        pl.BlockSpec((BN, D), lambda i: (i, 0)),
        pl.BlockSpec((4 * D, D), lambda i: (0, 0)),
        pl.BlockSpec((1, D), lambda i: (0, 0)),
    ],
    out_specs=pl.BlockSpec((BN, D), lambda i: (i, 0)),
    out_shape=jax.ShapeDtypeStruct((N, D), f32),
)


def kernel(x, edge_index, W1_0, W2_0, Wr_0, br_0, W1_1, W2_1, Wr_1, br_1,
           W1_2, W2_2, Wr_2, br_2, Wf, bf):
    src = edge_index[0]
    dst = edge_index[1]
    pad = EP - E
    padz = (jnp.arange(pad, dtype=jnp.int32) * 37) % N
    padd = DUMMY + (jnp.arange(pad, dtype=jnp.int32) % (NP - N))

    gidx = jnp.stack([
        jnp.concatenate([src, padz]),
        jnp.concatenate([dst + N, padz]),
    ]).reshape(NW * CH, CHUNK)
    sidx = jnp.stack([
        jnp.concatenate([dst, padd]),
        jnp.concatenate([src, padd]),
    ]).reshape(NW * CH, CHUNK)

    cnt = _counts(sidx)
    cd = cnt[0:N, 0:1]
    cs = cnt[NP:NP + N, 0:1]

    params = [(W1_0, W2_0, Wr_0, br_0), (W1_1, W2_1, Wr_1, br_1),
              (W1_2, W2_2, Wr_2, br_2)]
    hs = [x]
    h = x
    for (W1, W2, Wr, br) in params:
        a12, hr = _pre(h, jnp.stack([W1, W2]), Wr, br.reshape(1, D))
        accs = _scatter2(a12.reshape(2 * N, D), gidx, sidx)
        h = _comb(hr, accs.reshape(NC, NP, D), cd, cs)
        hs.append(h)

    return _fin(hs[0], hs[1], hs[2], hs[3], Wf, bf.reshape(1, D))
